# Initial kernel scaffold; baseline (speedup 1.0000x reference)
import jax
import jax.numpy as jnp
from jax.experimental import pallas as pl


def kernel(h, edge_index, dist, W, b, gamma, beta):
    raise NotImplementedError("write your pallas kernel here")


# speedup vs baseline: 1.8393x; 1.8393x over previous
import functools

import jax
import jax.numpy as jnp
from jax import lax
from jax.experimental import pallas as pl
from jax.experimental.pallas import tpu as pltpu
from jax.experimental.pallas import tpu_sc as plsc

N = 10000
E = 320000
D = 128
OUT = 128

NC = 2
NS = 16
NW = NC * NS
CPW = D // NW
CHUNK = 2048
NCHUNK = E // CHUNK
GRP = CHUNK // 16
NPAD = 10240


def _sc_body(ht_hbm, src_hbm, dst_hbm, dist_hbm, zacc_hbm, zdeg_hbm,
             out_hbm, hcol_v, acc_v, deg_v, src_v, dst_v, dist_v):
    c = lax.axis_index("c")
    s = lax.axis_index("s")
    w = c * NS + s

    pltpu.sync_copy(ht_hbm.at[pl.ds(w * CPW, CPW)], hcol_v)
    pltpu.sync_copy(zacc_hbm, acc_v)
    pltpu.sync_copy(zdeg_hbm, deg_v)

    ones16 = jnp.full((16,), 1.0, jnp.float32)
    zeros16i = jnp.full((16,), 0, jnp.int32)
    kfs = [jnp.full((16,), k, jnp.int32) for k in range(CPW)]

    def chunk_body(ci, carry):
        base = ci * CHUNK
        pltpu.sync_copy(src_hbm.at[pl.ds(base, CHUNK)], src_v)
        pltpu.sync_copy(dst_hbm.at[pl.ds(base, CHUNK)], dst_v)
        pltpu.sync_copy(dist_hbm.at[pl.ds(base, CHUNK)], dist_v)

        def grp_body(g, cc):
            sl = pl.ds(g * 16, 16)
            src16 = src_v[sl]
            dst16 = dst_v[sl]
            d16 = dist_v[sl]
            plsc.addupdate_scatter(deg_v, [zeros16i, dst16], ones16)
            for k in range(CPW):
                vals = plsc.load_gather(hcol_v, [kfs[k], src16])
                plsc.addupdate_scatter(acc_v, [kfs[k], dst16], vals * d16)
            return cc

        lax.fori_loop(0, GRP, grp_body, 0)
        return carry

    lax.fori_loop(0, NCHUNK, chunk_body, 0)

    def norm_body(i, carry):
        sl = pl.ds(i * 16, 16)
        dv = deg_v[0, sl]
        nv = jnp.where(dv > 0.0, 1.0 / dv, 0.0)
        for k in range(CPW):
            acc_v[k, sl] = acc_v[k, sl] * nv
        return carry

    lax.fori_loop(0, N // 16, norm_body, 0)

    pltpu.sync_copy(acc_v, out_hbm.at[pl.ds(w * CPW, CPW)])


_sc_aggregate = functools.partial(
    pl.kernel,
    out_type=jax.ShapeDtypeStruct((D, N), jnp.float32),
    mesh=plsc.VectorSubcoreMesh(core_axis_name="c", subcore_axis_name="s",
                                num_cores=NC, num_subcores=NS),
    compiler_params=pltpu.CompilerParams(needs_layout_passes=False),
    scratch_types=[
        pltpu.VMEM((CPW, N), jnp.float32),
        pltpu.VMEM((CPW, N), jnp.float32),
        pltpu.VMEM((1, NPAD), jnp.float32),
        pltpu.VMEM((CHUNK,), jnp.int32),
        pltpu.VMEM((CHUNK,), jnp.int32),
        pltpu.VMEM((CHUNK,), jnp.float32),
    ],
)(_sc_body)


TCR = 1000


def _tc_body(h_ref, ahn_ref, w_ref, b_ref, g_ref, be_ref, o_ref):
    hc = jnp.concatenate([h_ref[...], ahn_ref[...]], axis=1)
    z = lax.dot_general(hc, w_ref[...], (((1,), (1,)), ((), ())),
                        preferred_element_type=jnp.float32)
    z = z + b_ref[...]
    mu = jnp.mean(z, axis=-1, keepdims=True)
    var = jnp.mean((z - mu) * (z - mu), axis=-1, keepdims=True)
    zn = (z - mu) * lax.rsqrt(var + 1e-5)
    o_ref[...] = jnp.maximum(zn * g_ref[...] + be_ref[...], 0.0)


def _tc_finish(h, ahn, W, b, gamma, beta):
    return pl.pallas_call(
        _tc_body,
        grid=(N // TCR,),
        in_specs=[
            pl.BlockSpec((TCR, D), lambda i: (i, 0)),
            pl.BlockSpec((TCR, D), lambda i: (i, 0)),
            pl.BlockSpec((OUT, 2 * D), lambda i: (0, 0)),
            pl.BlockSpec((1, OUT), lambda i: (0, 0)),
            pl.BlockSpec((1, OUT), lambda i: (0, 0)),
            pl.BlockSpec((1, OUT), lambda i: (0, 0)),
        ],
        out_specs=pl.BlockSpec((TCR, OUT), lambda i: (i, 0)),
        out_shape=jax.ShapeDtypeStruct((N, OUT), jnp.float32),
    )(h, ahn, W, b, gamma, beta)


@jax.jit
def kernel(h, edge_index, dist, W, b, gamma, beta):
    src = edge_index[0]
    dst = edge_index[1]
    ht = h.T
    zacc = jnp.zeros((CPW, N), jnp.float32)
    zdeg = jnp.zeros((1, NPAD), jnp.float32)
    ahnt = _sc_aggregate(ht, src, dst, dist, zacc, zdeg)
    return _tc_finish(h, ahnt.T, W, b.reshape(1, OUT), gamma.reshape(1, OUT),
                      beta.reshape(1, OUT))


# speedup vs baseline: 4.5565x; 2.4772x over previous
import functools

import jax
import jax.numpy as jnp
from jax import lax
from jax.experimental import pallas as pl
from jax.experimental.pallas import tpu as pltpu
from jax.experimental.pallas import tpu_sc as plsc

N = 10000
E = 320000
D = 128
OUT = 128

NC = 2
NS = 16
NW = NC * NS
CPW = D // NW
CHUNK = 6400
NCHUNK = E // CHUNK
GRP = CHUNK // 16
NPAD = 10240


def _sc_body(ht_hbm, src_hbm, dst_hbm, dist_hbm, zacc_hbm, zdeg_hbm,
             out_hbm, hcol_v, acc_v, deg_v, src_v, dst_v, dist_v):
    c = lax.axis_index("c")
    s = lax.axis_index("s")
    w = c * NS + s

    pltpu.sync_copy(ht_hbm.at[pl.ds(w * CPW, CPW)], hcol_v)
    pltpu.sync_copy(zacc_hbm, acc_v)
    pltpu.sync_copy(zdeg_hbm, deg_v)

    ones16 = jnp.full((16,), 1.0, jnp.float32)
    zeros16i = jnp.full((16,), 0, jnp.int32)
    kfs = [jnp.full((16,), k, jnp.int32) for k in range(CPW)]

    def chunk_body(ci, carry):
        base = ci * CHUNK
        pltpu.sync_copy(src_hbm.at[pl.ds(base, CHUNK)], src_v)
        pltpu.sync_copy(dst_hbm.at[pl.ds(base, CHUNK)], dst_v)
        pltpu.sync_copy(dist_hbm.at[pl.ds(base, CHUNK)], dist_v)

        @plsc.parallel_loop(0, CHUNK, 16, unroll=8)
        def grp_body(e0):
            sl = pl.ds(e0, 16)
            src16 = src_v[sl]
            dst16 = dst_v[sl]
            d16 = dist_v[sl]
            plsc.addupdate_scatter(deg_v, [zeros16i, dst16], ones16)
            for k in range(CPW):
                vals = plsc.load_gather(hcol_v, [kfs[k], src16])
                plsc.addupdate_scatter(acc_v, [kfs[k], dst16], vals * d16)

        return carry

    lax.fori_loop(0, NCHUNK, chunk_body, 0)

    @plsc.parallel_loop(0, N, 16, unroll=4)
    def norm_body(v0):
        sl = pl.ds(v0, 16)
        dv = deg_v[0, sl]
        nv = jnp.where(dv > 0.0, 1.0 / dv, 0.0)
        for k in range(CPW):
            acc_v[k, sl] = acc_v[k, sl] * nv

    pltpu.sync_copy(acc_v, out_hbm.at[pl.ds(w * CPW, CPW)])


_sc_aggregate = functools.partial(
    pl.kernel,
    out_type=jax.ShapeDtypeStruct((D, N), jnp.float32),
    mesh=plsc.VectorSubcoreMesh(core_axis_name="c", subcore_axis_name="s",
                                num_cores=NC, num_subcores=NS),
    compiler_params=pltpu.CompilerParams(needs_layout_passes=False),
    scratch_types=[
        pltpu.VMEM((CPW, N), jnp.float32),
        pltpu.VMEM((CPW, N), jnp.float32),
        pltpu.VMEM((1, NPAD), jnp.float32),
        pltpu.VMEM((CHUNK,), jnp.int32),
        pltpu.VMEM((CHUNK,), jnp.int32),
        pltpu.VMEM((CHUNK,), jnp.float32),
    ],
)(_sc_body)


TCR = 1000


def _tc_body(h_ref, ahn_ref, w_ref, b_ref, g_ref, be_ref, o_ref):
    hc = jnp.concatenate([h_ref[...], ahn_ref[...]], axis=1)
    z = lax.dot_general(hc, w_ref[...], (((1,), (1,)), ((), ())),
                        preferred_element_type=jnp.float32)
    z = z + b_ref[...]
    mu = jnp.mean(z, axis=-1, keepdims=True)
    var = jnp.mean((z - mu) * (z - mu), axis=-1, keepdims=True)
    zn = (z - mu) * lax.rsqrt(var + 1e-5)
    o_ref[...] = jnp.maximum(zn * g_ref[...] + be_ref[...], 0.0)


def _tc_finish(h, ahn, W, b, gamma, beta):
    return pl.pallas_call(
        _tc_body,
        grid=(N // TCR,),
        in_specs=[
            pl.BlockSpec((TCR, D), lambda i: (i, 0)),
            pl.BlockSpec((TCR, D), lambda i: (i, 0)),
            pl.BlockSpec((OUT, 2 * D), lambda i: (0, 0)),
            pl.BlockSpec((1, OUT), lambda i: (0, 0)),
            pl.BlockSpec((1, OUT), lambda i: (0, 0)),
            pl.BlockSpec((1, OUT), lambda i: (0, 0)),
        ],
        out_specs=pl.BlockSpec((TCR, OUT), lambda i: (i, 0)),
        out_shape=jax.ShapeDtypeStruct((N, OUT), jnp.float32),
    )(h, ahn, W, b, gamma, beta)


@jax.jit
def kernel(h, edge_index, dist, W, b, gamma, beta):
    src = edge_index[0]
    dst = edge_index[1]
    ht = h.T
    zacc = jnp.zeros((CPW, N), jnp.float32)
    zdeg = jnp.zeros((1, NPAD), jnp.float32)
    ahnt = _sc_aggregate(ht, src, dst, dist, zacc, zdeg)
    return _tc_finish(h, ahnt.T, W, b.reshape(1, OUT), gamma.reshape(1, OUT),
                      beta.reshape(1, OUT))


# speedup vs baseline: 5.4828x; 1.2033x over previous
import functools

import jax
import jax.numpy as jnp
from jax import lax
from jax.experimental import pallas as pl
from jax.experimental.pallas import tpu as pltpu
from jax.experimental.pallas import tpu_sc as plsc

N = 10000
E = 320000
D = 128
OUT = 128

NC = 2
NS = 16
NW = NC * NS
CPW = D // NW
CHUNK = 10000
NCHUNK = E // CHUNK
NPAD = 10240


def _sc_body(ht_hbm, edata_hbm, zacc_hbm, zdeg_hbm,
             out_hbm, hcol_v, acc_v, deg_v, ebuf_v):
    c = lax.axis_index("c")
    s = lax.axis_index("s")
    w = c * NS + s

    pltpu.sync_copy(ht_hbm.at[pl.ds(w * CPW, CPW)], hcol_v)
    pltpu.sync_copy(zacc_hbm, acc_v)
    pltpu.sync_copy(zdeg_hbm, deg_v)

    ones16 = jnp.full((16,), 1.0, jnp.float32)
    zeros16i = jnp.full((16,), 0, jnp.int32)
    kfs = [jnp.full((16,), k, jnp.int32) for k in range(CPW)]

    def chunk_body(ci, carry):
        pltpu.sync_copy(edata_hbm.at[pl.ds(ci * 3 * CHUNK, 3 * CHUNK)],
                        ebuf_v)

        @plsc.parallel_loop(0, CHUNK, 16, unroll=8)
        def grp_body(e0):
            src16 = ebuf_v[pl.ds(e0, 16)]
            dst16 = ebuf_v[pl.ds(CHUNK + e0, 16)]
            d16 = plsc.bitcast(ebuf_v[pl.ds(2 * CHUNK + e0, 16)],
                               jnp.float32)
            plsc.addupdate_scatter(deg_v, [zeros16i, dst16], ones16)
            for k in range(CPW):
                vals = plsc.load_gather(hcol_v, [kfs[k], src16])
                plsc.addupdate_scatter(acc_v, [kfs[k], dst16], vals * d16)

        return carry

    lax.fori_loop(0, NCHUNK, chunk_body, 0)

    @plsc.parallel_loop(0, N, 16, unroll=4)
    def norm_body(v0):
        sl = pl.ds(v0, 16)
        dv = deg_v[0, sl]
        nv = jnp.where(dv > 0.0, 1.0 / dv, 0.0)
        for k in range(CPW):
            acc_v[k, sl] = acc_v[k, sl] * nv

    pltpu.sync_copy(acc_v, out_hbm.at[pl.ds(w * CPW, CPW)])


_sc_aggregate = functools.partial(
    pl.kernel,
    out_type=jax.ShapeDtypeStruct((D, N), jnp.float32),
    mesh=plsc.VectorSubcoreMesh(core_axis_name="c", subcore_axis_name="s",
                                num_cores=NC, num_subcores=NS),
    compiler_params=pltpu.CompilerParams(needs_layout_passes=False),
    scratch_types=[
        pltpu.VMEM((CPW, N), jnp.float32),
        pltpu.VMEM((CPW, N), jnp.float32),
        pltpu.VMEM((1, NPAD), jnp.float32),
        pltpu.VMEM((3 * CHUNK,), jnp.int32),
    ],
)(_sc_body)


TCR = 1000


def _tc_body(h_ref, ahn_ref, w_ref, b_ref, g_ref, be_ref, o_ref):
    hc = jnp.concatenate([h_ref[...], ahn_ref[...]], axis=1)
    z = lax.dot_general(hc, w_ref[...], (((1,), (1,)), ((), ())),
                        preferred_element_type=jnp.float32)
    z = z + b_ref[...]
    mu = jnp.mean(z, axis=-1, keepdims=True)
    var = jnp.mean((z - mu) * (z - mu), axis=-1, keepdims=True)
    zn = (z - mu) * lax.rsqrt(var + 1e-5)
    o_ref[...] = jnp.maximum(zn * g_ref[...] + be_ref[...], 0.0)


def _tc_finish(h, ahn, W, b, gamma, beta):
    return pl.pallas_call(
        _tc_body,
        grid=(N // TCR,),
        in_specs=[
            pl.BlockSpec((TCR, D), lambda i: (i, 0)),
            pl.BlockSpec((TCR, D), lambda i: (i, 0)),
            pl.BlockSpec((OUT, 2 * D), lambda i: (0, 0)),
            pl.BlockSpec((1, OUT), lambda i: (0, 0)),
            pl.BlockSpec((1, OUT), lambda i: (0, 0)),
            pl.BlockSpec((1, OUT), lambda i: (0, 0)),
        ],
        out_specs=pl.BlockSpec((TCR, OUT), lambda i: (i, 0)),
        out_shape=jax.ShapeDtypeStruct((N, OUT), jnp.float32),
    )(h, ahn, W, b, gamma, beta)


@jax.jit
def kernel(h, edge_index, dist, W, b, gamma, beta):
    dist_bits = lax.bitcast_convert_type(dist, jnp.int32)
    edata = (jnp.concatenate([edge_index, dist_bits[None]], axis=0)
             .reshape(3, NCHUNK, CHUNK).transpose(1, 0, 2).reshape(-1))
    ht = h.T
    zacc = jnp.zeros((CPW, N), jnp.float32)
    zdeg = jnp.zeros((1, NPAD), jnp.float32)
    ahnt = _sc_aggregate(ht, edata, zacc, zdeg)
    return _tc_finish(h, ahnt.T, W, b.reshape(1, OUT), gamma.reshape(1, OUT),
                      beta.reshape(1, OUT))
